# baseline (speedup 1.0000x reference)
import functools

import jax
import jax.numpy as jnp
from jax import lax
from jax.experimental import pallas as pl
from jax.experimental.pallas import tpu as pltpu
from jax.experimental.pallas import tpu_sc as plsc

N_NODES = 10000
N_FEAT = 256
N_Q = 64
N_GRAPHS = 128
N_EDGES_RAW = 160000
E_REAL = N_EDGES_RAW + N_NODES

NC = 2
NS = 16
LANES = 16

EDGE_WIN = 128
WIN_PER_TILE = 84
E_PAD = NS * WIN_PER_TILE * EDGE_WIN
DENOM_PAD = 10240
N_PAD = 10240
ROWS_PER_TILE = N_PAD // NS

R_BLK = 2000


def _sc_edge_kernel(src_hbm, dst_hbm, as_hbm, ad_hbm,
                    hq0_hbm, hq1_hbm, hq2_hbm, hq3_hbm,
                    oq0_hbm, oq1_hbm, oq2_hbm, oq3_hbm,
                    src_loc, dst_loc, ab_loc, as_v, ad_v, den_v, rows_a,
                    rows_b, zrow_v, dz_v, acc_sp, den_sp,
                    sem_ga, sem_gb, sem_s):
  cid = lax.axis_index("c")
  sid = lax.axis_index("s")
  zeros16 = jnp.zeros((LANES,), jnp.float32)
  lanes = lax.iota(jnp.int32, LANES)

  def _zrow(i, c):
    for cc in range(N_Q // LANES):
      zrow_v[i, pl.ds(cc * LANES, LANES)] = zeros16
    return c
  lax.fori_loop(0, 128, _zrow, 0)

  def _zdz(i, c):
    dz_v[pl.ds(pl.multiple_of(i * LANES, LANES), LANES)] = zeros16
    return c
  lax.fori_loop(0, 40, _zdz, 0)

  pltpu.sync_copy(dz_v, den_sp.at[pl.ds(sid * 640, 640)])

  pltpu.sync_copy(as_hbm, as_v)
  pltpu.sync_copy(ad_hbm, ad_v)
  pltpu.sync_copy(src_hbm.at[sid], src_loc)
  pltpu.sync_copy(dst_hbm.at[sid], dst_loc)

  plsc.subcore_barrier()

  edge_base = sid * WIN_PER_TILE * EDGE_WIN

  def _p1a(w, c):
    def _grp(j):
      off = pl.multiple_of(j * LANES, LANES)
      sv = src_loc[w, pl.ds(off, LANES)]
      dv = dst_loc[w, pl.ds(off, LANES)]
      s = plsc.load_gather(as_v, [sv])
      d = plsc.load_gather(ad_v, [dv])
      e = s + d
      e = jnp.where(e > 0.0, e, 0.2 * e)
      ex = jnp.exp(e)
      gid = edge_base + w * EDGE_WIN + j * LANES + lanes
      ex = jnp.where(gid < E_REAL, ex, 0.0)
      ab_loc[w, pl.ds(off, LANES)] = ex
    plsc.parallel_loop(0, EDGE_WIN // LANES, 1, unroll=8)(_grp)

    @pl.when(w >= 1)
    def _():
      pltpu.make_async_copy(
          ab_loc.at[w - 1], den_sp.at[dst_loc.at[w - 1]], sem_s).wait()
    pltpu.async_copy(ab_loc.at[w], den_sp.at[dst_loc.at[w]], sem_s, add=True)
    return c
  lax.fori_loop(0, WIN_PER_TILE, _p1a, 0)
  pltpu.make_async_copy(
      ab_loc.at[WIN_PER_TILE - 1],
      den_sp.at[dst_loc.at[WIN_PER_TILE - 1]], sem_s).wait()

  plsc.subcore_barrier()

  pltpu.sync_copy(den_sp, den_v)

  def _p1b(w, c):
    def _grp(j):
      off = pl.multiple_of(j * LANES, LANES)
      dv = dst_loc[w, pl.ds(off, LANES)]
      dn = plsc.load_gather(den_v, [dv])
      ex = ab_loc[w, pl.ds(off, LANES)]
      ab_loc[w, pl.ds(off, LANES)] = ex / (dn + 1e-16)
    plsc.parallel_loop(0, EDGE_WIN // LANES, 1, unroll=8)(_grp)
    return c
  lax.fori_loop(0, WIN_PER_TILE, _p1b, 0)

  def _p2(h_hbm):
    pltpu.async_copy(h_hbm.at[src_loc.at[0]], rows_a, sem_ga)

    def _pair(g, c):
      for b in range(2):
        buf, gsem = (rows_a, sem_ga) if b == 0 else (rows_b, sem_gb)
        obuf, ogsem = (rows_b, sem_gb) if b == 0 else (rows_a, sem_ga)
        w = 2 * g + b

        @pl.when(w + 1 < WIN_PER_TILE)
        def _():
          pltpu.async_copy(h_hbm.at[src_loc.at[w + 1]], obuf, ogsem)

        pltpu.make_async_copy(h_hbm.at[src_loc.at[w]], buf, gsem).wait()

        @plsc.parallel_loop(0, EDGE_WIN, 1, unroll=16)
        def _edge(j):
          wv = jnp.full((LANES,), w, jnp.int32)
          jv = jnp.full((LANES,), j, jnp.int32)
          av = plsc.load_gather(ab_loc, [wv, jv])
          for ccol in range(N_Q // LANES):
            sl = pl.ds(ccol * LANES, LANES)
            buf[j, sl] = buf[j, sl] * av

        pltpu.sync_copy(buf, acc_sp.at[dst_loc.at[w]], add=True)
      return c
    lax.fori_loop(0, WIN_PER_TILE // 2, _pair, 0)

  def _writeout(o_hbm):
    pltpu.sync_copy(acc_sp.at[pl.ds(sid * ROWS_PER_TILE, ROWS_PER_TILE)],
                    o_hbm.at[pl.ds(sid * ROWS_PER_TILE, ROWS_PER_TILE)])

  h_by_core = ((hq0_hbm, hq1_hbm), (hq2_hbm, hq3_hbm))
  o_by_core = ((oq0_hbm, oq1_hbm), (oq2_hbm, oq3_hbm))

  for q in range(2):
    for r in range(5):
      pltpu.sync_copy(zrow_v,
                      acc_sp.at[pl.ds(sid * ROWS_PER_TILE + r * 128, 128)])
    plsc.subcore_barrier()

    @pl.when(cid == 0)
    def _():
      _p2(h_by_core[0][q])

    @pl.when(cid == 1)
    def _():
      _p2(h_by_core[1][q])

    plsc.subcore_barrier()

    @pl.when(cid == 0)
    def _():
      _writeout(o_by_core[0][q])

    @pl.when(cid == 1)
    def _():
      _writeout(o_by_core[1][q])

    plsc.subcore_barrier()


def _sc_edge(src_p, dst_p, asv, adv, hq):
  mesh = plsc.VectorSubcoreMesh(core_axis_name="c", subcore_axis_name="s",
                                num_cores=NC, num_subcores=NS)
  f32 = jnp.float32
  return pl.kernel(
      _sc_edge_kernel,
      out_type=[jax.ShapeDtypeStruct((N_PAD, N_Q), f32) for _ in range(4)],
      mesh=mesh,
      scratch_types=[
          pltpu.VMEM((WIN_PER_TILE, EDGE_WIN), jnp.int32),
          pltpu.VMEM((WIN_PER_TILE, EDGE_WIN), jnp.int32),
          pltpu.VMEM((WIN_PER_TILE, EDGE_WIN), f32),
          pltpu.VMEM((N_NODES,), f32),
          pltpu.VMEM((N_NODES,), f32),
          pltpu.VMEM((DENOM_PAD,), f32),
          pltpu.VMEM((EDGE_WIN, N_Q), f32),
          pltpu.VMEM((EDGE_WIN, N_Q), f32),
          pltpu.VMEM((128, N_Q), f32),
          pltpu.VMEM((640,), f32),
          pltpu.VMEM_SHARED((N_PAD, N_Q), f32),
          pltpu.VMEM_SHARED((DENOM_PAD,), f32),
          pltpu.SemaphoreType.DMA,
          pltpu.SemaphoreType.DMA,
          pltpu.SemaphoreType.DMA,
      ],
      compiler_params=pltpu.CompilerParams(needs_layout_passes=False,
                                           use_tc_tiling_on_sc=False),
  )(src_p, dst_p, asv, adv, *hq)


def _gelu(x):
  return 0.5 * x * (1.0 + lax.erf(x * 0.7071067811865476))


def _tc_layer_body(first, *refs):
  if first:
    (x_ref, wt_ref, avec_ref, b_ref,
     o0_ref, o1_ref, o2_ref, o3_ref, aa_ref) = refs
    A = x_ref[...]
  else:
    (q0_ref, q1_ref, q2_ref, q3_ref, wt_ref, avec_ref, b_ref,
     o0_ref, o1_ref, o2_ref, o3_ref, aa_ref) = refs
    A = jnp.concatenate(
        [q0_ref[...], q1_ref[...], q2_ref[...], q3_ref[...]], axis=1)
    A = _gelu(A + b_ref[...])
  H = jnp.dot(A, wt_ref[...], preferred_element_type=jnp.float32)
  o0_ref[...] = H[:, 0 * N_Q:1 * N_Q]
  o1_ref[...] = H[:, 1 * N_Q:2 * N_Q]
  o2_ref[...] = H[:, 2 * N_Q:3 * N_Q]
  o3_ref[...] = H[:, 3 * N_Q:4 * N_Q]
  aa_ref[...] = jnp.dot(H, avec_ref[...], preferred_element_type=jnp.float32)


def _make_qspec(c):
  return pl.BlockSpec((R_BLK, N_Q), lambda i, c=c: (i, c))


def _tc_layer(inq, Wt, avec, b, first):
  f32 = jnp.float32
  grid = N_NODES // R_BLK
  if first:
    data_specs = [pl.BlockSpec((R_BLK, N_FEAT), lambda i: (i, 0))]
  else:
    data_specs = [_make_qspec(0) for _ in range(4)]
  in_specs = data_specs + [
      pl.BlockSpec((N_FEAT, N_FEAT), lambda i: (0, 0)),
      pl.BlockSpec((N_FEAT, 2), lambda i: (0, 0)),
      pl.BlockSpec((1, N_FEAT), lambda i: (0, 0)),
  ]
  return pl.pallas_call(
      functools.partial(_tc_layer_body, first),
      grid=(grid,),
      in_specs=in_specs,
      out_specs=[_make_qspec(0) for _ in range(4)] +
                [pl.BlockSpec((R_BLK, 2), lambda i: (i, 0))],
      out_shape=[jax.ShapeDtypeStruct((N_NODES, N_Q), f32)
                 for _ in range(4)] +
                [jax.ShapeDtypeStruct((N_NODES, 2), f32)],
  )(*inq, Wt, avec, b)


def _pool_body(q0_ref, q1_ref, q2_ref, q3_ref, batch_ref, b3_ref, l1wt_ref,
               l1b_ref, l2wt_ref, l2b_ref, out_ref, pooled_acc, cnt_acc):
  i = pl.program_id(0)

  @pl.when(i == 0)
  def _():
    pooled_acc[...] = jnp.zeros_like(pooled_acc)
    cnt_acc[...] = jnp.zeros_like(cnt_acc)

  A = jnp.concatenate(
      [q0_ref[...], q1_ref[...], q2_ref[...], q3_ref[...]], axis=1)
  A = _gelu(A + b3_ref[...])
  bt = batch_ref[...]
  gids = lax.broadcasted_iota(jnp.int32, (1, N_GRAPHS), 1)
  B = (bt == gids).astype(jnp.float32)
  pooled_acc[...] += lax.dot_general(
      B, A, (((0,), (0,)), ((), ())), preferred_element_type=jnp.float32)
  ones = jnp.ones((R_BLK, 1), jnp.float32)
  cnt_acc[...] += lax.dot_general(
      B, ones, (((0,), (0,)), ((), ())), preferred_element_type=jnp.float32)

  @pl.when(i == pl.num_programs(0) - 1)
  def _():
    mean = pooled_acc[...] / jnp.maximum(cnt_acc[...], 1.0)
    z = jnp.dot(mean, l1wt_ref[...],
                preferred_element_type=jnp.float32) + l1b_ref[...]
    z = _gelu(z)
    out_ref[...] = jnp.dot(z, l2wt_ref[...],
                           preferred_element_type=jnp.float32) + l2b_ref[...]


def _pool_mlp(oq, batch2d, b3, l1wt, l1b, l2wt, l2b):
  f32 = jnp.float32
  grid = N_NODES // R_BLK
  nh2 = N_FEAT // 2
  return pl.pallas_call(
      _pool_body,
      grid=(grid,),
      in_specs=[_make_qspec(0) for _ in range(4)] + [
          pl.BlockSpec((R_BLK, 1), lambda i: (i, 0)),
          pl.BlockSpec((1, N_FEAT), lambda i: (0, 0)),
          pl.BlockSpec((N_FEAT, nh2), lambda i: (0, 0)),
          pl.BlockSpec((1, nh2), lambda i: (0, 0)),
          pl.BlockSpec((nh2, 1), lambda i: (0, 0)),
          pl.BlockSpec((1, 1), lambda i: (0, 0)),
      ],
      out_specs=pl.BlockSpec((N_GRAPHS, 1), lambda i: (0, 0)),
      out_shape=jax.ShapeDtypeStruct((N_GRAPHS, 1), f32),
      scratch_shapes=[
          pltpu.VMEM((N_GRAPHS, N_FEAT), f32),
          pltpu.VMEM((N_GRAPHS, 1), f32),
      ],
      compiler_params=pltpu.CompilerParams(
          dimension_semantics=("arbitrary",)),
  )(*oq, batch2d, b3, l1wt, l1b, l2wt, l2b)


def kernel(x, edge_index, batch, W1, a_src1, a_dst1, b1, W2, a_src2, a_dst2,
           b2, W3, a_src3, a_dst3, b3, lin1_W, lin1_b, lin2_W, lin2_b):
  f32 = jnp.float32
  loop = jnp.arange(N_NODES, dtype=edge_index.dtype)
  pad = jnp.arange(E_PAD - E_REAL, dtype=jnp.int32) % N_NODES
  src_p = jnp.concatenate([edge_index[0], loop, pad]).reshape(
      NS, WIN_PER_TILE, EDGE_WIN)
  dst_p = jnp.concatenate([edge_index[1], loop, pad]).reshape(
      NS, WIN_PER_TILE, EDGE_WIN)

  zb = jnp.zeros((1, N_FEAT), f32)
  av1 = jnp.stack([a_src1, a_dst1], axis=1)
  av2 = jnp.stack([a_src2, a_dst2], axis=1)
  av3 = jnp.stack([a_src3, a_dst3], axis=1)

  *hq, aa = _tc_layer((x,), W1.T, av1, zb, first=True)
  oq = _sc_edge(src_p, dst_p, aa[:, 0], aa[:, 1], hq)

  *hq, aa = _tc_layer(oq, W2.T, av2, b1.reshape(1, -1), first=False)
  oq = _sc_edge(src_p, dst_p, aa[:, 0], aa[:, 1], hq)

  *hq, aa = _tc_layer(oq, W3.T, av3, b2.reshape(1, -1), first=False)
  oq = _sc_edge(src_p, dst_p, aa[:, 0], aa[:, 1], hq)

  return _pool_mlp(oq, batch.reshape(-1, 1).astype(jnp.int32),
                   b3.reshape(1, -1), lin1_W.T, lin1_b.reshape(1, -1),
                   lin2_W.T, lin2_b.reshape(1, -1))


# speedup vs baseline: 1.0071x; 1.0071x over previous
import functools

import jax
import jax.numpy as jnp
from jax import lax
from jax.experimental import pallas as pl
from jax.experimental.pallas import tpu as pltpu
from jax.experimental.pallas import tpu_sc as plsc

N_NODES = 10000
N_FEAT = 256
N_Q = 64
N_GRAPHS = 128
N_EDGES_RAW = 160000
E_REAL = N_EDGES_RAW + N_NODES

NC = 2
NS = 16
LANES = 16

EDGE_WIN = 128
WIN_PER_TILE = 84
E_PAD = NS * WIN_PER_TILE * EDGE_WIN
DENOM_PAD = 10240
N_PAD = 10240
ROWS_PER_TILE = N_PAD // NS

R_BLK = 2000


def _sc_edge_kernel(src_hbm, dst_hbm, as_hbm, ad_hbm,
                    hq0_hbm, hq1_hbm, hq2_hbm, hq3_hbm,
                    oq0_hbm, oq1_hbm, oq2_hbm, oq3_hbm,
                    src_loc, dst_loc, ab_loc, as_v, ad_v, den_v, rows_a,
                    rows_b, zrow_v, dz_v, acc_sp, den_sp,
                    sem_ga, sem_gb, sem_s):
  cid = lax.axis_index("c")
  sid = lax.axis_index("s")
  zeros16 = jnp.zeros((LANES,), jnp.float32)
  lanes = lax.iota(jnp.int32, LANES)

  def _zrow(i, c):
    for cc in range(N_Q // LANES):
      zrow_v[i, pl.ds(cc * LANES, LANES)] = zeros16
    return c
  lax.fori_loop(0, 128, _zrow, 0)

  def _zdz(i, c):
    dz_v[pl.ds(pl.multiple_of(i * LANES, LANES), LANES)] = zeros16
    return c
  lax.fori_loop(0, 40, _zdz, 0)

  pltpu.sync_copy(dz_v, den_sp.at[pl.ds(sid * 640, 640)])

  pltpu.sync_copy(as_hbm, as_v)
  pltpu.sync_copy(ad_hbm, ad_v)
  pltpu.sync_copy(src_hbm.at[sid], src_loc)
  pltpu.sync_copy(dst_hbm.at[sid], dst_loc)

  plsc.subcore_barrier()

  edge_base = sid * WIN_PER_TILE * EDGE_WIN

  def _p1a(w, c):
    def _grp(j):
      off = pl.multiple_of(j * LANES, LANES)
      sv = src_loc[w, pl.ds(off, LANES)]
      dv = dst_loc[w, pl.ds(off, LANES)]
      s = plsc.load_gather(as_v, [sv])
      d = plsc.load_gather(ad_v, [dv])
      e = s + d
      e = jnp.where(e > 0.0, e, 0.2 * e)
      ex = jnp.exp(e)
      gid = edge_base + w * EDGE_WIN + j * LANES + lanes
      ex = jnp.where(gid < E_REAL, ex, 0.0)
      ab_loc[w, pl.ds(off, LANES)] = ex
    plsc.parallel_loop(0, EDGE_WIN // LANES, 1, unroll=8)(_grp)

    @pl.when(w >= 1)
    def _():
      pltpu.make_async_copy(
          ab_loc.at[w - 1], den_sp.at[dst_loc.at[w - 1]], sem_s).wait()
    pltpu.async_copy(ab_loc.at[w], den_sp.at[dst_loc.at[w]], sem_s, add=True)
    return c
  lax.fori_loop(0, WIN_PER_TILE, _p1a, 0)
  pltpu.make_async_copy(
      ab_loc.at[WIN_PER_TILE - 1],
      den_sp.at[dst_loc.at[WIN_PER_TILE - 1]], sem_s).wait()

  plsc.subcore_barrier()

  pltpu.sync_copy(den_sp, den_v)

  def _p1b(w, c):
    def _grp(j):
      off = pl.multiple_of(j * LANES, LANES)
      dv = dst_loc[w, pl.ds(off, LANES)]
      dn = plsc.load_gather(den_v, [dv])
      ex = ab_loc[w, pl.ds(off, LANES)]
      ab_loc[w, pl.ds(off, LANES)] = ex / (dn + 1e-16)
    plsc.parallel_loop(0, EDGE_WIN // LANES, 1, unroll=8)(_grp)
    return c
  lax.fori_loop(0, WIN_PER_TILE, _p1b, 0)

  def _p2(h_hbm):
    pltpu.async_copy(h_hbm.at[src_loc.at[0]], rows_a, sem_ga)

    def _pair(g, c):
      for b in range(2):
        buf, gsem = (rows_a, sem_ga) if b == 0 else (rows_b, sem_gb)
        obuf, ogsem = (rows_b, sem_gb) if b == 0 else (rows_a, sem_ga)
        w = 2 * g + b

        @pl.when(w + 1 < WIN_PER_TILE)
        def _():
          pltpu.async_copy(h_hbm.at[src_loc.at[w + 1]], obuf, ogsem)

        pltpu.make_async_copy(h_hbm.at[src_loc.at[w]], buf, gsem).wait()

        @plsc.parallel_loop(0, EDGE_WIN, 1, unroll=8)
        def _edge(j):
          wv = jnp.full((LANES,), w, jnp.int32)
          jv = jnp.full((LANES,), j, jnp.int32)
          av = plsc.load_gather(ab_loc, [wv, jv])
          for ccol in range(N_Q // LANES):
            sl = pl.ds(ccol * LANES, LANES)
            buf[j, sl] = buf[j, sl] * av

        pltpu.sync_copy(buf, acc_sp.at[dst_loc.at[w]], add=True)
      return c
    lax.fori_loop(0, WIN_PER_TILE // 2, _pair, 0)

  def _writeout(o_hbm):
    pltpu.sync_copy(acc_sp.at[pl.ds(sid * ROWS_PER_TILE, ROWS_PER_TILE)],
                    o_hbm.at[pl.ds(sid * ROWS_PER_TILE, ROWS_PER_TILE)])

  h_by_core = ((hq0_hbm, hq1_hbm), (hq2_hbm, hq3_hbm))
  o_by_core = ((oq0_hbm, oq1_hbm), (oq2_hbm, oq3_hbm))

  for q in range(2):
    for r in range(5):
      pltpu.sync_copy(zrow_v,
                      acc_sp.at[pl.ds(sid * ROWS_PER_TILE + r * 128, 128)])
    plsc.subcore_barrier()

    @pl.when(cid == 0)
    def _():
      _p2(h_by_core[0][q])

    @pl.when(cid == 1)
    def _():
      _p2(h_by_core[1][q])

    plsc.subcore_barrier()

    @pl.when(cid == 0)
    def _():
      _writeout(o_by_core[0][q])

    @pl.when(cid == 1)
    def _():
      _writeout(o_by_core[1][q])

    plsc.subcore_barrier()


def _sc_edge(src_p, dst_p, asv, adv, hq):
  mesh = plsc.VectorSubcoreMesh(core_axis_name="c", subcore_axis_name="s",
                                num_cores=NC, num_subcores=NS)
  f32 = jnp.float32
  return pl.kernel(
      _sc_edge_kernel,
      out_type=[jax.ShapeDtypeStruct((N_PAD, N_Q), f32) for _ in range(4)],
      mesh=mesh,
      scratch_types=[
          pltpu.VMEM((WIN_PER_TILE, EDGE_WIN), jnp.int32),
          pltpu.VMEM((WIN_PER_TILE, EDGE_WIN), jnp.int32),
          pltpu.VMEM((WIN_PER_TILE, EDGE_WIN), f32),
          pltpu.VMEM((N_NODES,), f32),
          pltpu.VMEM((N_NODES,), f32),
          pltpu.VMEM((DENOM_PAD,), f32),
          pltpu.VMEM((EDGE_WIN, N_Q), f32),
          pltpu.VMEM((EDGE_WIN, N_Q), f32),
          pltpu.VMEM((128, N_Q), f32),
          pltpu.VMEM((640,), f32),
          pltpu.VMEM_SHARED((N_PAD, N_Q), f32),
          pltpu.VMEM_SHARED((DENOM_PAD,), f32),
          pltpu.SemaphoreType.DMA,
          pltpu.SemaphoreType.DMA,
          pltpu.SemaphoreType.DMA,
      ],
      compiler_params=pltpu.CompilerParams(needs_layout_passes=False,
                                           use_tc_tiling_on_sc=False),
  )(src_p, dst_p, asv, adv, *hq)


def _gelu(x):
  return 0.5 * x * (1.0 + lax.erf(x * 0.7071067811865476))


def _tc_layer_body(first, *refs):
  if first:
    (x_ref, wt_ref, avec_ref, b_ref,
     o0_ref, o1_ref, o2_ref, o3_ref, aa_ref) = refs
    A = x_ref[...]
  else:
    (q0_ref, q1_ref, q2_ref, q3_ref, wt_ref, avec_ref, b_ref,
     o0_ref, o1_ref, o2_ref, o3_ref, aa_ref) = refs
    A = jnp.concatenate(
        [q0_ref[...], q1_ref[...], q2_ref[...], q3_ref[...]], axis=1)
    A = _gelu(A + b_ref[...])
  H = jnp.dot(A, wt_ref[...], preferred_element_type=jnp.float32)
  o0_ref[...] = H[:, 0 * N_Q:1 * N_Q]
  o1_ref[...] = H[:, 1 * N_Q:2 * N_Q]
  o2_ref[...] = H[:, 2 * N_Q:3 * N_Q]
  o3_ref[...] = H[:, 3 * N_Q:4 * N_Q]
  aa_ref[...] = jnp.dot(H, avec_ref[...], preferred_element_type=jnp.float32)


def _make_qspec(c):
  return pl.BlockSpec((R_BLK, N_Q), lambda i, c=c: (i, c))


def _tc_layer(inq, Wt, avec, b, first):
  f32 = jnp.float32
  grid = N_NODES // R_BLK
  if first:
    data_specs = [pl.BlockSpec((R_BLK, N_FEAT), lambda i: (i, 0))]
  else:
    data_specs = [_make_qspec(0) for _ in range(4)]
  in_specs = data_specs + [
      pl.BlockSpec((N_FEAT, N_FEAT), lambda i: (0, 0)),
      pl.BlockSpec((N_FEAT, 2), lambda i: (0, 0)),
      pl.BlockSpec((1, N_FEAT), lambda i: (0, 0)),
  ]
  return pl.pallas_call(
      functools.partial(_tc_layer_body, first),
      grid=(grid,),
      in_specs=in_specs,
      out_specs=[_make_qspec(0) for _ in range(4)] +
                [pl.BlockSpec((R_BLK, 2), lambda i: (i, 0))],
      out_shape=[jax.ShapeDtypeStruct((N_NODES, N_Q), f32)
                 for _ in range(4)] +
                [jax.ShapeDtypeStruct((N_NODES, 2), f32)],
  )(*inq, Wt, avec, b)


def _pool_body(q0_ref, q1_ref, q2_ref, q3_ref, batch_ref, b3_ref, l1wt_ref,
               l1b_ref, l2wt_ref, l2b_ref, out_ref, pooled_acc, cnt_acc):
  i = pl.program_id(0)

  @pl.when(i == 0)
  def _():
    pooled_acc[...] = jnp.zeros_like(pooled_acc)
    cnt_acc[...] = jnp.zeros_like(cnt_acc)

  A = jnp.concatenate(
      [q0_ref[...], q1_ref[...], q2_ref[...], q3_ref[...]], axis=1)
  A = _gelu(A + b3_ref[...])
  bt = batch_ref[...]
  gids = lax.broadcasted_iota(jnp.int32, (1, N_GRAPHS), 1)
  B = (bt == gids).astype(jnp.float32)
  pooled_acc[...] += lax.dot_general(
      B, A, (((0,), (0,)), ((), ())), preferred_element_type=jnp.float32)
  ones = jnp.ones((R_BLK, 1), jnp.float32)
  cnt_acc[...] += lax.dot_general(
      B, ones, (((0,), (0,)), ((), ())), preferred_element_type=jnp.float32)

  @pl.when(i == pl.num_programs(0) - 1)
  def _():
    mean = pooled_acc[...] / jnp.maximum(cnt_acc[...], 1.0)
    z = jnp.dot(mean, l1wt_ref[...],
                preferred_element_type=jnp.float32) + l1b_ref[...]
    z = _gelu(z)
    out_ref[...] = jnp.dot(z, l2wt_ref[...],
                           preferred_element_type=jnp.float32) + l2b_ref[...]


def _pool_mlp(oq, batch2d, b3, l1wt, l1b, l2wt, l2b):
  f32 = jnp.float32
  grid = N_NODES // R_BLK
  nh2 = N_FEAT // 2
  return pl.pallas_call(
      _pool_body,
      grid=(grid,),
      in_specs=[_make_qspec(0) for _ in range(4)] + [
          pl.BlockSpec((R_BLK, 1), lambda i: (i, 0)),
          pl.BlockSpec((1, N_FEAT), lambda i: (0, 0)),
          pl.BlockSpec((N_FEAT, nh2), lambda i: (0, 0)),
          pl.BlockSpec((1, nh2), lambda i: (0, 0)),
          pl.BlockSpec((nh2, 1), lambda i: (0, 0)),
          pl.BlockSpec((1, 1), lambda i: (0, 0)),
      ],
      out_specs=pl.BlockSpec((N_GRAPHS, 1), lambda i: (0, 0)),
      out_shape=jax.ShapeDtypeStruct((N_GRAPHS, 1), f32),
      scratch_shapes=[
          pltpu.VMEM((N_GRAPHS, N_FEAT), f32),
          pltpu.VMEM((N_GRAPHS, 1), f32),
      ],
      compiler_params=pltpu.CompilerParams(
          dimension_semantics=("arbitrary",)),
  )(*oq, batch2d, b3, l1wt, l1b, l2wt, l2b)


def kernel(x, edge_index, batch, W1, a_src1, a_dst1, b1, W2, a_src2, a_dst2,
           b2, W3, a_src3, a_dst3, b3, lin1_W, lin1_b, lin2_W, lin2_b):
  f32 = jnp.float32
  loop = jnp.arange(N_NODES, dtype=edge_index.dtype)
  pad = jnp.arange(E_PAD - E_REAL, dtype=jnp.int32) % N_NODES
  src_p = jnp.concatenate([edge_index[0], loop, pad]).reshape(
      NS, WIN_PER_TILE, EDGE_WIN)
  dst_p = jnp.concatenate([edge_index[1], loop, pad]).reshape(
      NS, WIN_PER_TILE, EDGE_WIN)

  zb = jnp.zeros((1, N_FEAT), f32)
  av1 = jnp.stack([a_src1, a_dst1], axis=1)
  av2 = jnp.stack([a_src2, a_dst2], axis=1)
  av3 = jnp.stack([a_src3, a_dst3], axis=1)

  *hq, aa = _tc_layer((x,), W1.T, av1, zb, first=True)
  oq = _sc_edge(src_p, dst_p, aa[:, 0], aa[:, 1], hq)

  *hq, aa = _tc_layer(oq, W2.T, av2, b1.reshape(1, -1), first=False)
  oq = _sc_edge(src_p, dst_p, aa[:, 0], aa[:, 1], hq)

  *hq, aa = _tc_layer(oq, W3.T, av3, b2.reshape(1, -1), first=False)
  oq = _sc_edge(src_p, dst_p, aa[:, 0], aa[:, 1], hq)

  return _pool_mlp(oq, batch.reshape(-1, 1).astype(jnp.int32),
                   b3.reshape(1, -1), lin1_W.T, lin1_b.reshape(1, -1),
                   lin2_W.T, lin2_b.reshape(1, -1))
